# baseline (speedup 1.0000x reference)
import functools
import math

import jax
import jax.numpy as jnp
from jax import lax
from jax.experimental import pallas as pl
from jax.experimental.pallas import tpu as pltpu
from jax.experimental.pallas import tpu_sc as plsc

_M = 256


def _routing_body(K, M, TP, kf_ref, pos_ref, meta_ref):
    R, C = kf_ref.shape
    kf = kf_ref[...]
    row = lax.broadcasted_iota(jnp.int32, (C, C), 0)
    col = lax.broadcasted_iota(jnp.int32, (C, C), 1)
    l_incl = (row <= col).astype(jnp.float32)
    rr = lax.broadcasted_iota(jnp.int32, (R, R), 0)
    cc = lax.broadcasted_iota(jnp.int32, (R, R), 1)
    l_strict = (cc < rr).astype(jnp.float32)

    ranks = []
    masks = []
    tiles_cum = []
    total_tiles = jnp.int32(0)
    pos = jnp.zeros((R, C), jnp.int32)
    for k in range(K):
        m = (kf == k)
        x = m.astype(jnp.float32)
        pref = lax.dot_general(x, l_incl, (((1,), (0,)), ((), ())),
                               preferred_element_type=jnp.float32)
        rowtot = pref[:, C - 1:C]
        rowoff = lax.dot_general(l_strict, rowtot, (((1,), (0,)), ((), ())),
                                 preferred_element_type=jnp.float32)
        rank = (pref - 1.0 + rowoff).astype(jnp.int32)
        cnt = jnp.sum(x).astype(jnp.int32)
        ntiles = (cnt + (M - 1)) // M
        start = total_tiles
        total_tiles = total_tiles + ntiles
        tiles_cum.append(total_tiles)
        pos = pos + jnp.where(m, start * M + rank, 0)
    pos_ref[...] = pos

    ic = lax.broadcasted_iota(jnp.int32, (1, C), 1)
    eot = jnp.zeros((1, C), jnp.int32)
    for k in range(K):
        eot = eot + (ic >= tiles_cum[k]).astype(jnp.int32)
    eot = jnp.minimum(eot, K - 1)
    meta = jnp.where(ic < TP, eot, 0) + jnp.where(ic == 64, total_tiles, 0)
    meta_ref[...] = meta


def _mlp_body(e_ref, u_ref, vals_a_ref, vals_b_ref, bcol_ref, w1_ref, b1_ref,
              w2_ref, b2_ref, out_ref):
    @pl.when(pl.program_id(0) < u_ref[0])
    def _():
        v = vals_a_ref[0] + vals_b_ref[0]
        yt = bcol_ref[0] * v
        fft = jnp.concatenate([jnp.sin(yt), jnp.cos(yt)], axis=0)
        h = lax.dot_general(fft, w1_ref[0], (((0,), (0,)), ((), ())),
                            preferred_element_type=jnp.float32)
        h = h + b1_ref[0]
        h = 0.5 * h * (1.0 + lax.erf(h * (1.0 / math.sqrt(2.0))))
        out_ref[...] = jnp.dot(h, w2_ref[0],
                               preferred_element_type=jnp.float32) + b2_ref[0]


def kernel(values, kinds, Bmat, kind_emb, W1, b1, W2, b2):
    N, S, _ = values.shape
    T = N * S
    K, d = kind_emb.shape
    B = Bmat.shape[1]
    Bp = ((B + 31) // 32) * 32
    M = _M
    TP = T // M + K - 1
    Tpad = TP * M

    f32 = jnp.float32
    NC, NS = 2, 16
    NW = NC * NS
    tpw = T // NW

    kf2d = kinds.reshape(NW, tpw).astype(jnp.int32)
    pos2d, meta = pl.pallas_call(
        functools.partial(_routing_body, K, M, TP),
        grid=(1,),
        in_specs=[pl.BlockSpec((NW, tpw), lambda i: (0, 0))],
        out_specs=[pl.BlockSpec((NW, tpw), lambda i: (0, 0)),
                   pl.BlockSpec((1, tpw), lambda i: (0, 0))],
        out_shape=[jax.ShapeDtypeStruct((NW, tpw), jnp.int32),
                   jax.ShapeDtypeStruct((1, tpw), jnp.int32)],
    )(kf2d)
    expert_of_tile = meta[0, :TP]
    used_tiles = meta[0, 64:65]

    zpad = jnp.zeros((K, Bp - B, d), f32)
    W1p = jnp.concatenate([W1[:, :B], zpad, W1[:, B:], zpad], axis=1)
    b1r = b1.reshape(K, 1, d)
    b2r = (b2 + kind_emb).reshape(K, 1, d)
    bcol = jnp.pad((2.0 * math.pi) * Bmat[0], (0, Bp - B)).reshape(1, Bp, 1)

    vals2d = values.reshape(NW, tpw)

    mesh = plsc.VectorSubcoreMesh(core_axis_name="c", subcore_axis_name="s")

    zpw = Tpad // NS

    @functools.partial(
        pl.kernel, mesh=mesh,
        out_type=jax.ShapeDtypeStruct((NC, Tpad), f32),
        scratch_types=[
            pltpu.VMEM((tpw,), jnp.int32),
            pltpu.VMEM((tpw,), f32),
            pltpu.VMEM((zpw,), f32),
            pltpu.VMEM_SHARED((Tpad,), f32),
        ],
    )
    def scatter_vals(vals_hbm, pos_hbm, out_hbm, idx_v, val_v, zero_v, shared):
        cid = lax.axis_index("c")
        sid = lax.axis_index("s")
        wid = sid * NC + cid

        def zbody(i, carry):
            zero_v[pl.ds(i * 16, 16)] = jnp.zeros((16,), f32)
            return carry

        lax.fori_loop(0, zpw // 16, zbody, 0)
        pltpu.sync_copy(zero_v, shared.at[pl.ds(sid * zpw, zpw)])
        pltpu.sync_copy(pos_hbm.at[wid], idx_v)
        pltpu.sync_copy(vals_hbm.at[wid], val_v)
        plsc.subcore_barrier()
        pltpu.sync_copy(val_v, shared.at[idx_v])
        plsc.subcore_barrier()

        @pl.when(sid == 0)
        def _():
            pltpu.sync_copy(shared, out_hbm.at[cid])

    vals_sorted2 = scatter_vals(vals2d, pos2d)
    vals_sorted = vals_sorted2.reshape(NC * TP, 1, M)

    grid_spec = pltpu.PrefetchScalarGridSpec(
        num_scalar_prefetch=2,
        grid=(TP,),
        in_specs=[
            pl.BlockSpec((1, 1, M), lambda i, e, u: (i, 0, 0)),
            pl.BlockSpec((1, 1, M), lambda i, e, u: (TP + i, 0, 0)),
            pl.BlockSpec((1, Bp, 1), lambda i, e, u: (0, 0, 0)),
            pl.BlockSpec((1, 2 * Bp, d), lambda i, e, u: (e[i], 0, 0)),
            pl.BlockSpec((1, 1, d), lambda i, e, u: (e[i], 0, 0)),
            pl.BlockSpec((1, d, d), lambda i, e, u: (e[i], 0, 0)),
            pl.BlockSpec((1, 1, d), lambda i, e, u: (e[i], 0, 0)),
        ],
        out_specs=pl.BlockSpec((M, d), lambda i, e, u: (i, 0)),
    )
    out_sorted = pl.pallas_call(
        _mlp_body,
        grid_spec=grid_spec,
        out_shape=jax.ShapeDtypeStruct((Tpad, d), f32),
        compiler_params=pltpu.CompilerParams(
            dimension_semantics=("arbitrary",)),
    )(expert_of_tile, used_tiles, vals_sorted, vals_sorted, bcol, W1p,
      b1r, W2, b2r)

    C = 32

    @functools.partial(
        pl.kernel, mesh=mesh,
        out_type=jax.ShapeDtypeStruct((T, d), f32),
        scratch_types=[
            pltpu.VMEM((C,), jnp.int32),
            pltpu.VMEM((C, d), f32),
            pltpu.SemaphoreType.DMA,
        ],
    )
    def gather_rows(table_hbm, pos_hbm, out_hbm, idx_v, rows_v, sem):
        wid = lax.axis_index("s") * NC + lax.axis_index("c")
        base = wid * tpw

        def body(c, carry):
            b = base + c * C
            pltpu.sync_copy(pos_hbm.at[pl.ds(b, C)], idx_v)
            pltpu.async_copy(table_hbm.at[idx_v], rows_v, sem).wait()
            pltpu.sync_copy(rows_v, out_hbm.at[pl.ds(b, C)])
            return carry

        lax.fori_loop(0, tpw // C, body, 0)

    out = gather_rows(out_sorted, pos2d.reshape(T))
    return out.reshape(N, S, d)


# speedup vs baseline: 1.0886x; 1.0886x over previous
import functools
import math

import jax
import jax.numpy as jnp
from jax import lax
from jax.experimental import pallas as pl
from jax.experimental.pallas import tpu as pltpu
from jax.experimental.pallas import tpu_sc as plsc

_M = 512


def _routing_body(K, M, TP, kf_ref, pos_ref, meta_ref):
    R, C = kf_ref.shape
    kf = kf_ref[...]
    row = lax.broadcasted_iota(jnp.int32, (C, C), 0)
    col = lax.broadcasted_iota(jnp.int32, (C, C), 1)
    l_incl = (row <= col).astype(jnp.float32)
    rr = lax.broadcasted_iota(jnp.int32, (R, R), 0)
    cc = lax.broadcasted_iota(jnp.int32, (R, R), 1)
    l_strict = (cc < rr).astype(jnp.float32)

    ranks = []
    masks = []
    tiles_cum = []
    total_tiles = jnp.int32(0)
    pos = jnp.zeros((R, C), jnp.int32)
    for k in range(K):
        m = (kf == k)
        x = m.astype(jnp.float32)
        pref = lax.dot_general(x, l_incl, (((1,), (0,)), ((), ())),
                               preferred_element_type=jnp.float32)
        rowtot = pref[:, C - 1:C]
        rowoff = lax.dot_general(l_strict, rowtot, (((1,), (0,)), ((), ())),
                                 preferred_element_type=jnp.float32)
        rank = (pref - 1.0 + rowoff).astype(jnp.int32)
        cnt = jnp.sum(x).astype(jnp.int32)
        ntiles = (cnt + (M - 1)) // M
        start = total_tiles
        total_tiles = total_tiles + ntiles
        tiles_cum.append(total_tiles)
        pos = pos + jnp.where(m, start * M + rank, 0)
    pos_ref[...] = pos

    ic = lax.broadcasted_iota(jnp.int32, (1, C), 1)
    eot = jnp.zeros((1, C), jnp.int32)
    for k in range(K):
        eot = eot + (ic >= tiles_cum[k]).astype(jnp.int32)
    eot = jnp.minimum(eot, K - 1)
    meta = jnp.where(ic < TP, eot, 0) + jnp.where(ic == 64, total_tiles, 0)
    meta_ref[...] = meta


def _mlp_body(e_ref, u_ref, vals_a_ref, vals_b_ref, bcol_ref, w1_ref, b1_ref,
              w2_ref, b2_ref, out_ref):
    @pl.when(pl.program_id(0) < u_ref[0])
    def _():
        v = vals_a_ref[0] + vals_b_ref[0]
        yt = bcol_ref[0] * v
        fft = jnp.concatenate([jnp.sin(yt), jnp.cos(yt)], axis=0)
        h = lax.dot_general(fft, w1_ref[0], (((0,), (0,)), ((), ())),
                            preferred_element_type=jnp.float32)
        h = h + b1_ref[0]
        h = 0.5 * h * (1.0 + lax.erf(h * (1.0 / math.sqrt(2.0))))
        out_ref[...] = jnp.dot(h, w2_ref[0],
                               preferred_element_type=jnp.float32) + b2_ref[0]


def kernel(values, kinds, Bmat, kind_emb, W1, b1, W2, b2):
    N, S, _ = values.shape
    T = N * S
    K, d = kind_emb.shape
    B = Bmat.shape[1]
    Bp = ((B + 31) // 32) * 32
    M = _M
    TP = T // M + K - 1
    Tpad = TP * M

    f32 = jnp.float32
    NC, NS = 2, 16
    NW = NC * NS
    tpw = T // NW

    kf2d = kinds.reshape(NW, tpw).astype(jnp.int32)
    pos2d, meta = pl.pallas_call(
        functools.partial(_routing_body, K, M, TP),
        grid=(1,),
        in_specs=[pl.BlockSpec((NW, tpw), lambda i: (0, 0))],
        out_specs=[pl.BlockSpec((NW, tpw), lambda i: (0, 0)),
                   pl.BlockSpec((1, tpw), lambda i: (0, 0))],
        out_shape=[jax.ShapeDtypeStruct((NW, tpw), jnp.int32),
                   jax.ShapeDtypeStruct((1, tpw), jnp.int32)],
    )(kf2d)
    expert_of_tile = meta[0, :TP]
    used_tiles = meta[0, 64:65]

    zpad = jnp.zeros((K, Bp - B, d), f32)
    W1p = jnp.concatenate([W1[:, :B], zpad, W1[:, B:], zpad], axis=1)
    b1r = b1.reshape(K, 1, d)
    b2r = (b2 + kind_emb).reshape(K, 1, d)
    bcol = jnp.pad((2.0 * math.pi) * Bmat[0], (0, Bp - B)).reshape(1, Bp, 1)

    vals2d = values.reshape(NW, tpw)

    mesh = plsc.VectorSubcoreMesh(core_axis_name="c", subcore_axis_name="s")

    zpw = Tpad // NS

    @functools.partial(
        pl.kernel, mesh=mesh,
        out_type=jax.ShapeDtypeStruct((NC, Tpad), f32),
        scratch_types=[
            pltpu.VMEM((tpw,), jnp.int32),
            pltpu.VMEM((tpw,), f32),
            pltpu.VMEM((zpw,), f32),
            pltpu.VMEM_SHARED((Tpad,), f32),
        ],
    )
    def scatter_vals(vals_hbm, pos_hbm, out_hbm, idx_v, val_v, zero_v, shared):
        cid = lax.axis_index("c")
        sid = lax.axis_index("s")
        wid = sid * NC + cid

        def zbody(i, carry):
            zero_v[pl.ds(i * 16, 16)] = jnp.zeros((16,), f32)
            return carry

        lax.fori_loop(0, zpw // 16, zbody, 0)
        pltpu.sync_copy(zero_v, shared.at[pl.ds(sid * zpw, zpw)])
        pltpu.sync_copy(pos_hbm.at[wid], idx_v)
        pltpu.sync_copy(vals_hbm.at[wid], val_v)
        plsc.subcore_barrier()
        pltpu.sync_copy(val_v, shared.at[idx_v])
        plsc.subcore_barrier()

        @pl.when(sid == 0)
        def _():
            pltpu.sync_copy(shared, out_hbm.at[cid])

    vals_sorted2 = scatter_vals(vals2d, pos2d)
    vals_sorted = vals_sorted2.reshape(NC * TP, 1, M)

    grid_spec = pltpu.PrefetchScalarGridSpec(
        num_scalar_prefetch=2,
        grid=(TP,),
        in_specs=[
            pl.BlockSpec((1, 1, M), lambda i, e, u: (i, 0, 0)),
            pl.BlockSpec((1, 1, M), lambda i, e, u: (TP + i, 0, 0)),
            pl.BlockSpec((1, Bp, 1), lambda i, e, u: (0, 0, 0)),
            pl.BlockSpec((1, 2 * Bp, d), lambda i, e, u: (e[i], 0, 0)),
            pl.BlockSpec((1, 1, d), lambda i, e, u: (e[i], 0, 0)),
            pl.BlockSpec((1, d, d), lambda i, e, u: (e[i], 0, 0)),
            pl.BlockSpec((1, 1, d), lambda i, e, u: (e[i], 0, 0)),
        ],
        out_specs=pl.BlockSpec((M, d), lambda i, e, u: (i, 0)),
    )
    out_sorted = pl.pallas_call(
        _mlp_body,
        grid_spec=grid_spec,
        out_shape=jax.ShapeDtypeStruct((Tpad, d), f32),
        compiler_params=pltpu.CompilerParams(
            dimension_semantics=("arbitrary",)),
    )(expert_of_tile, used_tiles, vals_sorted, vals_sorted, bcol, W1p,
      b1r, W2, b2r)

    C = 32

    @functools.partial(
        pl.kernel, mesh=mesh,
        out_type=jax.ShapeDtypeStruct((T, d), f32),
        scratch_types=[
            pltpu.VMEM((C,), jnp.int32),
            pltpu.VMEM((C, d), f32),
            pltpu.SemaphoreType.DMA,
        ],
    )
    def gather_rows(table_hbm, pos_hbm, out_hbm, idx_v, rows_v, sem):
        wid = lax.axis_index("s") * NC + lax.axis_index("c")
        base = wid * tpw

        def body(c, carry):
            b = base + c * C
            pltpu.sync_copy(pos_hbm.at[pl.ds(b, C)], idx_v)
            pltpu.async_copy(table_hbm.at[idx_v], rows_v, sem).wait()
            pltpu.sync_copy(rows_v, out_hbm.at[pl.ds(b, C)])
            return carry

        lax.fori_loop(0, tpw // C, body, 0)

    out = gather_rows(out_sorted, pos2d.reshape(T))
    return out.reshape(N, S, d)


# speedup vs baseline: 1.0896x; 1.0009x over previous
import functools
import math

import jax
import jax.numpy as jnp
from jax import lax
from jax.experimental import pallas as pl
from jax.experimental.pallas import tpu as pltpu
from jax.experimental.pallas import tpu_sc as plsc

_M = 512


def _live(i, u_ref):
    return jnp.minimum(i, u_ref[0])


def _routing_body(K, M, TP, kf_ref, pos_ref, meta_ref):
    R, C = kf_ref.shape
    kf = kf_ref[...]
    row = lax.broadcasted_iota(jnp.int32, (C, C), 0)
    col = lax.broadcasted_iota(jnp.int32, (C, C), 1)
    l_incl = (row <= col).astype(jnp.float32)
    rr = lax.broadcasted_iota(jnp.int32, (R, R), 0)
    cc = lax.broadcasted_iota(jnp.int32, (R, R), 1)
    l_strict = (cc < rr).astype(jnp.float32)

    ranks = []
    masks = []
    tiles_cum = []
    total_tiles = jnp.int32(0)
    pos = jnp.zeros((R, C), jnp.int32)
    for k in range(K):
        m = (kf == k)
        x = m.astype(jnp.float32)
        pref = lax.dot_general(x, l_incl, (((1,), (0,)), ((), ())),
                               preferred_element_type=jnp.float32)
        rowtot = pref[:, C - 1:C]
        rowoff = lax.dot_general(l_strict, rowtot, (((1,), (0,)), ((), ())),
                                 preferred_element_type=jnp.float32)
        rank = (pref - 1.0 + rowoff).astype(jnp.int32)
        cnt = jnp.sum(x).astype(jnp.int32)
        ntiles = (cnt + (M - 1)) // M
        start = total_tiles
        total_tiles = total_tiles + ntiles
        tiles_cum.append(total_tiles)
        pos = pos + jnp.where(m, start * M + rank, 0)
    pos_ref[...] = pos

    ic = lax.broadcasted_iota(jnp.int32, (1, C), 1)
    eot = jnp.zeros((1, C), jnp.int32)
    for k in range(K):
        eot = eot + (ic >= tiles_cum[k]).astype(jnp.int32)
    eot = jnp.minimum(eot, K - 1)
    meta = jnp.where(ic < TP, eot, 0) + jnp.where(ic == 64, total_tiles, 0)
    meta_ref[...] = meta


def _mlp_body(e_ref, u_ref, vals_a_ref, vals_b_ref, bcol_ref, w1_ref, b1_ref,
              w2_ref, b2_ref, out_ref):
    @pl.when(pl.program_id(0) < u_ref[0])
    def _():
        v = vals_a_ref[0] + vals_b_ref[0]
        yt = bcol_ref[0] * v
        fft = jnp.concatenate([jnp.sin(yt), jnp.cos(yt)], axis=0)
        h = lax.dot_general(fft, w1_ref[0], (((0,), (0,)), ((), ())),
                            preferred_element_type=jnp.float32)
        h = h + b1_ref[0]
        h = 0.5 * h * (1.0 + lax.erf(h * (1.0 / math.sqrt(2.0))))
        out_ref[...] = jnp.dot(h, w2_ref[0],
                               preferred_element_type=jnp.float32) + b2_ref[0]


def kernel(values, kinds, Bmat, kind_emb, W1, b1, W2, b2):
    N, S, _ = values.shape
    T = N * S
    K, d = kind_emb.shape
    B = Bmat.shape[1]
    Bp = ((B + 31) // 32) * 32
    M = _M
    TP = T // M + K - 1
    Tpad = TP * M

    f32 = jnp.float32
    NC, NS = 2, 16
    NW = NC * NS
    tpw = T // NW

    kf2d = kinds.reshape(NW, tpw).astype(jnp.int32)
    pos2d, meta = pl.pallas_call(
        functools.partial(_routing_body, K, M, TP),
        grid=(1,),
        in_specs=[pl.BlockSpec((NW, tpw), lambda i: (0, 0))],
        out_specs=[pl.BlockSpec((NW, tpw), lambda i: (0, 0)),
                   pl.BlockSpec((1, tpw), lambda i: (0, 0))],
        out_shape=[jax.ShapeDtypeStruct((NW, tpw), jnp.int32),
                   jax.ShapeDtypeStruct((1, tpw), jnp.int32)],
    )(kf2d)
    expert_of_tile = meta[0, :TP]
    used_tiles = meta[0, 64:65]

    zpad = jnp.zeros((K, Bp - B, d), f32)
    W1p = jnp.concatenate([W1[:, :B], zpad, W1[:, B:], zpad], axis=1)
    b1r = b1.reshape(K, 1, d)
    b2r = (b2 + kind_emb).reshape(K, 1, d)
    bcol = jnp.pad((2.0 * math.pi) * Bmat[0], (0, Bp - B)).reshape(1, Bp, 1)

    vals2d = values.reshape(NW, tpw)

    mesh = plsc.VectorSubcoreMesh(core_axis_name="c", subcore_axis_name="s")

    zpw = Tpad // NS

    @functools.partial(
        pl.kernel, mesh=mesh,
        out_type=jax.ShapeDtypeStruct((NC, Tpad), f32),
        scratch_types=[
            pltpu.VMEM((tpw,), jnp.int32),
            pltpu.VMEM((tpw,), f32),
            pltpu.VMEM((zpw,), f32),
            pltpu.VMEM_SHARED((Tpad,), f32),
        ],
    )
    def scatter_vals(vals_hbm, pos_hbm, out_hbm, idx_v, val_v, zero_v, shared):
        cid = lax.axis_index("c")
        sid = lax.axis_index("s")
        wid = sid * NC + cid

        def zbody(i, carry):
            zero_v[pl.ds(i * 16, 16)] = jnp.zeros((16,), f32)
            return carry

        lax.fori_loop(0, zpw // 16, zbody, 0)
        pltpu.sync_copy(zero_v, shared.at[pl.ds(sid * zpw, zpw)])
        pltpu.sync_copy(pos_hbm.at[wid], idx_v)
        pltpu.sync_copy(vals_hbm.at[wid], val_v)
        plsc.subcore_barrier()
        pltpu.sync_copy(val_v, shared.at[idx_v])
        plsc.subcore_barrier()

        @pl.when(sid == 0)
        def _():
            pltpu.sync_copy(shared, out_hbm.at[cid])

    vals_sorted2 = scatter_vals(vals2d, pos2d)
    vals_sorted = vals_sorted2.reshape(NC * TP, 1, M)

    grid_spec = pltpu.PrefetchScalarGridSpec(
        num_scalar_prefetch=2,
        grid=(TP,),
        in_specs=[
            pl.BlockSpec((1, 1, M),
                         lambda i, e, u: (_live(i, u), 0, 0)),
            pl.BlockSpec((1, 1, M),
                         lambda i, e, u: (TP + _live(i, u), 0, 0)),
            pl.BlockSpec((1, Bp, 1), lambda i, e, u: (0, 0, 0)),
            pl.BlockSpec((1, 2 * Bp, d), lambda i, e, u: (e[i], 0, 0)),
            pl.BlockSpec((1, 1, d), lambda i, e, u: (e[i], 0, 0)),
            pl.BlockSpec((1, d, d), lambda i, e, u: (e[i], 0, 0)),
            pl.BlockSpec((1, 1, d), lambda i, e, u: (e[i], 0, 0)),
        ],
        out_specs=pl.BlockSpec((M, d), lambda i, e, u: (_live(i, u), 0)),
    )
    out_sorted = pl.pallas_call(
        _mlp_body,
        grid_spec=grid_spec,
        out_shape=jax.ShapeDtypeStruct((Tpad, d), f32),
        compiler_params=pltpu.CompilerParams(
            dimension_semantics=("arbitrary",)),
    )(expert_of_tile, used_tiles, vals_sorted, vals_sorted, bcol, W1p,
      b1r, W2, b2r)

    C = 32

    @functools.partial(
        pl.kernel, mesh=mesh,
        out_type=jax.ShapeDtypeStruct((T, d), f32),
        scratch_types=[
            pltpu.VMEM((C,), jnp.int32),
            pltpu.VMEM((C, d), f32),
            pltpu.SemaphoreType.DMA,
        ],
    )
    def gather_rows(table_hbm, pos_hbm, out_hbm, idx_v, rows_v, sem):
        wid = lax.axis_index("s") * NC + lax.axis_index("c")
        base = wid * tpw

        def body(c, carry):
            b = base + c * C
            pltpu.sync_copy(pos_hbm.at[pl.ds(b, C)], idx_v)
            pltpu.async_copy(table_hbm.at[idx_v], rows_v, sem).wait()
            pltpu.sync_copy(rows_v, out_hbm.at[pl.ds(b, C)])
            return carry

        lax.fori_loop(0, tpw // C, body, 0)

    out = gather_rows(out_sorted, pos2d.reshape(T))
    return out.reshape(N, S, d)


# speedup vs baseline: 1.0993x; 1.0089x over previous
import functools
import math

import jax
import jax.numpy as jnp
from jax import lax
from jax.experimental import pallas as pl
from jax.experimental.pallas import tpu as pltpu
from jax.experimental.pallas import tpu_sc as plsc

_M = 512


def _live(i, u_ref):
    return jnp.minimum(i, u_ref[0])


def _routing_body(K, M, TP, kf_ref, pos_ref, meta_ref):
    R, C = kf_ref.shape
    kf = kf_ref[...]
    row = lax.broadcasted_iota(jnp.int32, (C, C), 0)
    col = lax.broadcasted_iota(jnp.int32, (C, C), 1)
    l_incl = (row <= col).astype(jnp.float32)
    rr = lax.broadcasted_iota(jnp.int32, (R, R), 0)
    cc = lax.broadcasted_iota(jnp.int32, (R, R), 1)
    l_strict = (cc < rr).astype(jnp.float32)

    ranks = []
    masks = []
    tiles_cum = []
    total_tiles = jnp.int32(0)
    pos = jnp.zeros((R, C), jnp.int32)
    for k in range(K):
        m = (kf == k)
        x = m.astype(jnp.float32)
        pref = lax.dot_general(x, l_incl, (((1,), (0,)), ((), ())),
                               preferred_element_type=jnp.float32)
        rowtot = pref[:, C - 1:C]
        rowoff = lax.dot_general(l_strict, rowtot, (((1,), (0,)), ((), ())),
                                 preferred_element_type=jnp.float32)
        rank = (pref - 1.0 + rowoff).astype(jnp.int32)
        cnt = jnp.sum(x).astype(jnp.int32)
        ntiles = (cnt + (M - 1)) // M
        start = total_tiles
        total_tiles = total_tiles + ntiles
        tiles_cum.append(total_tiles)
        pos = pos + jnp.where(m, start * M + rank, 0)
    pos_ref[...] = pos

    ic = lax.broadcasted_iota(jnp.int32, (1, C), 1)
    eot = jnp.zeros((1, C), jnp.int32)
    for k in range(K):
        eot = eot + (ic >= tiles_cum[k]).astype(jnp.int32)
    eot = jnp.minimum(eot, K - 1)
    meta = jnp.where(ic < TP, eot, 0) + jnp.where(ic == 64, total_tiles, 0)
    meta_ref[...] = meta


def _mlp_body(e_ref, u_ref, vals_a_ref, vals_b_ref, bcol_ref, w1_ref, b1_ref,
              w2_ref, b2_ref, out_ref):
    @pl.when(pl.program_id(0) < u_ref[0])
    def _():
        v = vals_a_ref[0] + vals_b_ref[0]
        yt = bcol_ref[0] * v
        fft = jnp.concatenate([jnp.sin(yt), jnp.cos(yt)], axis=0)
        h = lax.dot_general(fft, w1_ref[0], (((0,), (0,)), ((), ())),
                            preferred_element_type=jnp.float32)
        h = h + b1_ref[0]
        h = 0.5 * h * (1.0 + lax.erf(h * (1.0 / math.sqrt(2.0))))
        out_ref[...] = jnp.dot(h, w2_ref[0],
                               preferred_element_type=jnp.float32) + b2_ref[0]


def kernel(values, kinds, Bmat, kind_emb, W1, b1, W2, b2):
    N, S, _ = values.shape
    T = N * S
    K, d = kind_emb.shape
    B = Bmat.shape[1]
    Bp = ((B + 31) // 32) * 32
    M = _M
    TP = T // M + K - 1
    Tpad = TP * M

    f32 = jnp.float32
    NC, NS = 2, 16
    NW = NC * NS
    tpw = T // NW

    kf2d = kinds.reshape(NW, tpw).astype(jnp.int32)
    pos2d, meta = pl.pallas_call(
        functools.partial(_routing_body, K, M, TP),
        grid=(1,),
        in_specs=[pl.BlockSpec((NW, tpw), lambda i: (0, 0))],
        out_specs=[pl.BlockSpec((NW, tpw), lambda i: (0, 0)),
                   pl.BlockSpec((1, tpw), lambda i: (0, 0))],
        out_shape=[jax.ShapeDtypeStruct((NW, tpw), jnp.int32),
                   jax.ShapeDtypeStruct((1, tpw), jnp.int32)],
    )(kf2d)
    expert_of_tile = meta[0, :TP]
    used_tiles = meta[0, 64:65]

    zpad = jnp.zeros((K, Bp - B, d), f32)
    W1p = jnp.concatenate([W1[:, :B], zpad, W1[:, B:], zpad], axis=1)
    b1r = b1.reshape(K, 1, d)
    b2r = (b2 + kind_emb).reshape(K, 1, d)
    bcol = jnp.pad((2.0 * math.pi) * Bmat[0], (0, Bp - B)).reshape(1, Bp, 1)

    vals2d = values.reshape(NW, tpw)

    mesh = plsc.VectorSubcoreMesh(core_axis_name="c", subcore_axis_name="s")

    zpw = Tpad // NS

    @functools.partial(
        pl.kernel, mesh=mesh,
        out_type=jax.ShapeDtypeStruct((NC, Tpad), f32),
        scratch_types=[
            pltpu.VMEM((tpw,), jnp.int32),
            pltpu.VMEM((tpw,), f32),
            pltpu.VMEM((zpw,), f32),
            pltpu.VMEM_SHARED((Tpad,), f32),
        ],
    )
    def scatter_vals(vals_hbm, pos_hbm, out_hbm, idx_v, val_v, zero_v, shared):
        cid = lax.axis_index("c")
        sid = lax.axis_index("s")
        wid = sid * NC + cid

        def zbody(i, carry):
            zero_v[pl.ds(i * 16, 16)] = jnp.zeros((16,), f32)
            return carry

        lax.fori_loop(0, zpw // 16, zbody, 0)
        pltpu.sync_copy(zero_v, shared.at[pl.ds(sid * zpw, zpw)])
        pltpu.sync_copy(pos_hbm.at[wid], idx_v)
        pltpu.sync_copy(vals_hbm.at[wid], val_v)
        plsc.subcore_barrier()
        pltpu.sync_copy(val_v, shared.at[idx_v])
        plsc.subcore_barrier()

        @pl.when(sid == 0)
        def _():
            pltpu.sync_copy(shared, out_hbm.at[cid])

    vals_sorted2 = scatter_vals(vals2d, pos2d)
    vals_sorted = vals_sorted2.reshape(NC * TP, 1, M)

    grid_spec = pltpu.PrefetchScalarGridSpec(
        num_scalar_prefetch=2,
        grid=(TP,),
        in_specs=[
            pl.BlockSpec((1, 1, M),
                         lambda i, e, u: (_live(i, u), 0, 0)),
            pl.BlockSpec((1, 1, M),
                         lambda i, e, u: (TP + _live(i, u), 0, 0)),
            pl.BlockSpec((1, Bp, 1), lambda i, e, u: (0, 0, 0)),
            pl.BlockSpec((1, 2 * Bp, d), lambda i, e, u: (e[i], 0, 0)),
            pl.BlockSpec((1, 1, d), lambda i, e, u: (e[i], 0, 0)),
            pl.BlockSpec((1, d, d), lambda i, e, u: (e[i], 0, 0)),
            pl.BlockSpec((1, 1, d), lambda i, e, u: (e[i], 0, 0)),
        ],
        out_specs=pl.BlockSpec((M, d), lambda i, e, u: (_live(i, u), 0)),
    )
    out_sorted = pl.pallas_call(
        _mlp_body,
        grid_spec=grid_spec,
        out_shape=jax.ShapeDtypeStruct((Tpad, d), f32),
        compiler_params=pltpu.CompilerParams(
            dimension_semantics=("arbitrary",)),
    )(expert_of_tile, used_tiles, vals_sorted, vals_sorted, bcol, W1p,
      b1r, W2, b2r)

    C = 32
    NCH = tpw // C

    @functools.partial(
        pl.kernel, mesh=mesh,
        out_type=jax.ShapeDtypeStruct((T, d), f32),
        scratch_types=[
            pltpu.VMEM((C,), jnp.int32),
            pltpu.VMEM((C,), jnp.int32),
            pltpu.VMEM((C, d), f32),
            pltpu.VMEM((C, d), f32),
            pltpu.SemaphoreType.DMA,
            pltpu.SemaphoreType.DMA,
            pltpu.SemaphoreType.DMA,
            pltpu.SemaphoreType.DMA,
            pltpu.SemaphoreType.DMA,
            pltpu.SemaphoreType.DMA,
        ],
    )
    def gather_rows(table_hbm, pos_hbm, out_hbm, i0, i1, r0, r1,
                    si0, si1, sg0, sg1, sw0, sw1):
        wid = lax.axis_index("s") * NC + lax.axis_index("c")
        base = wid * tpw
        idx = [i0, i1]
        rows = [r0, r1]
        sis = [si0, si1]
        sgs = [sg0, sg1]
        sws = [sw0, sw1]

        di = [None] * NCH
        dg = [None] * NCH
        dw = [None] * NCH
        di[0] = pltpu.async_copy(pos_hbm.at[pl.ds(base, C)], idx[0], sis[0])
        for c in range(NCH):
            b = c % 2
            if c + 1 < NCH:
                di[c + 1] = pltpu.async_copy(
                    pos_hbm.at[pl.ds(base + (c + 1) * C, C)],
                    idx[(c + 1) % 2], sis[(c + 1) % 2])
            di[c].wait()
            if c >= 2:
                dw[c - 2].wait()
            dg[c] = pltpu.async_copy(table_hbm.at[idx[b]], rows[b], sgs[b])
            dg[c].wait()
            dw[c] = pltpu.async_copy(rows[b], out_hbm.at[pl.ds(base + c * C, C)],
                                     sws[b])
        for c in range(max(0, NCH - 2), NCH):
            dw[c].wait()

    out = gather_rows(out_sorted, pos2d.reshape(T))
    return out.reshape(N, S, d)


# speedup vs baseline: 1.1871x; 1.0799x over previous
import functools
import math

import jax
import jax.numpy as jnp
from jax import lax
from jax.experimental import pallas as pl
from jax.experimental.pallas import tpu as pltpu
from jax.experimental.pallas import tpu_sc as plsc

_M = 512


def _live(i, m_ref):
    return jnp.minimum(i, m_ref[0, 64])


def _routing_body(K, M, TP, kf_ref, pos_ref, meta_ref):
    R, C = kf_ref.shape
    kf = kf_ref[...]
    row = lax.broadcasted_iota(jnp.int32, (C, C), 0)
    col = lax.broadcasted_iota(jnp.int32, (C, C), 1)
    l_incl = (row <= col).astype(jnp.float32)
    rr = lax.broadcasted_iota(jnp.int32, (R, R), 0)
    cc = lax.broadcasted_iota(jnp.int32, (R, R), 1)
    l_strict = (cc < rr).astype(jnp.float32)

    ranks = []
    masks = []
    tiles_cum = []
    total_tiles = jnp.int32(0)
    pos = jnp.zeros((R, C), jnp.int32)
    for k in range(K):
        m = (kf == k)
        x = m.astype(jnp.float32)
        pref = lax.dot_general(x, l_incl, (((1,), (0,)), ((), ())),
                               preferred_element_type=jnp.float32)
        rowtot = pref[:, C - 1:C]
        rowoff = lax.dot_general(l_strict, rowtot, (((1,), (0,)), ((), ())),
                                 preferred_element_type=jnp.float32)
        rank = (pref - 1.0 + rowoff).astype(jnp.int32)
        cnt = jnp.sum(x).astype(jnp.int32)
        ntiles = (cnt + (M - 1)) // M
        start = total_tiles
        total_tiles = total_tiles + ntiles
        tiles_cum.append(total_tiles)
        pos = pos + jnp.where(m, start * M + rank, 0)
    pos_ref[...] = pos

    ic = lax.broadcasted_iota(jnp.int32, (1, C), 1)
    eot = jnp.zeros((1, C), jnp.int32)
    for k in range(K):
        eot = eot + (ic >= tiles_cum[k]).astype(jnp.int32)
    eot = jnp.minimum(eot, K - 1)
    meta = jnp.where(ic < TP, eot, 0) + jnp.where(ic == 64, total_tiles, 0)
    meta_ref[...] = meta


def _mlp_body(m_ref, vals_a_ref, vals_b_ref, bcol_ref, w1_ref, b1_ref,
              w2_ref, b2_ref, out_ref):
    @pl.when(pl.program_id(0) < m_ref[0, 64])
    def _():
        v = vals_a_ref[0] + vals_b_ref[0]
        yt = bcol_ref[0] * v
        fft = jnp.concatenate([jnp.sin(yt), jnp.cos(yt)], axis=0)
        h = lax.dot_general(fft, w1_ref[0], (((0,), (0,)), ((), ())),
                            preferred_element_type=jnp.float32)
        h = h + b1_ref[0]
        h = 0.5 * h * (1.0 + lax.erf(h * (1.0 / math.sqrt(2.0))))
        out_ref[...] = jnp.dot(h, w2_ref[0],
                               preferred_element_type=jnp.float32) + b2_ref[0]


def kernel(values, kinds, Bmat, kind_emb, W1, b1, W2, b2):
    N, S, _ = values.shape
    T = N * S
    K, d = kind_emb.shape
    B = Bmat.shape[1]
    Bp = ((B + 7) // 8) * 8
    M = _M
    TP = T // M + K - 1
    Tpad = TP * M

    f32 = jnp.float32
    NC, NS = 2, 16
    NW = NC * NS
    tpw = T // NW

    kf2d = kinds.reshape(NW, tpw).astype(jnp.int32)
    pos2d, meta = pl.pallas_call(
        functools.partial(_routing_body, K, M, TP),
        grid=(1,),
        in_specs=[pl.BlockSpec((NW, tpw), lambda i: (0, 0))],
        out_specs=[pl.BlockSpec((NW, tpw), lambda i: (0, 0)),
                   pl.BlockSpec((1, tpw), lambda i: (0, 0))],
        out_shape=[jax.ShapeDtypeStruct((NW, tpw), jnp.int32),
                   jax.ShapeDtypeStruct((1, tpw), jnp.int32)],
    )(kf2d)

    if Bp == B:
        W1p = W1
        bcol = ((2.0 * math.pi) * Bmat[0]).reshape(1, B, 1)
    else:
        zpad = jnp.zeros((K, Bp - B, d), f32)
        W1p = jnp.concatenate([W1[:, :B], zpad, W1[:, B:], zpad], axis=1)
        bcol = jnp.pad((2.0 * math.pi) * Bmat[0], (0, Bp - B)).reshape(1, Bp, 1)
    b1r = b1.reshape(K, 1, d)
    b2r = (b2 + kind_emb).reshape(K, 1, d)

    vals2d = values.reshape(NW, tpw)

    mesh = plsc.VectorSubcoreMesh(core_axis_name="c", subcore_axis_name="s")

    zpw = Tpad // NS

    @functools.partial(
        pl.kernel, mesh=mesh,
        out_type=jax.ShapeDtypeStruct((NC, Tpad), f32),
        scratch_types=[
            pltpu.VMEM((tpw,), jnp.int32),
            pltpu.VMEM((tpw,), f32),
            pltpu.VMEM((zpw,), f32),
            pltpu.VMEM_SHARED((Tpad,), f32),
        ],
    )
    def scatter_vals(vals_hbm, pos_hbm, out_hbm, idx_v, val_v, zero_v, shared):
        cid = lax.axis_index("c")
        sid = lax.axis_index("s")
        wid = sid * NC + cid

        def zbody(i, carry):
            zero_v[pl.ds(i * 16, 16)] = jnp.zeros((16,), f32)
            return carry

        lax.fori_loop(0, zpw // 16, zbody, 0)
        pltpu.sync_copy(zero_v, shared.at[pl.ds(sid * zpw, zpw)])
        pltpu.sync_copy(pos_hbm.at[wid], idx_v)
        pltpu.sync_copy(vals_hbm.at[wid], val_v)
        plsc.subcore_barrier()
        pltpu.sync_copy(val_v, shared.at[idx_v])
        plsc.subcore_barrier()

        @pl.when(sid == 0)
        def _():
            pltpu.sync_copy(shared, out_hbm.at[cid])

    vals_sorted2 = scatter_vals(vals2d, pos2d)
    vals_sorted = vals_sorted2.reshape(NC * TP, 1, M)

    grid_spec = pltpu.PrefetchScalarGridSpec(
        num_scalar_prefetch=1,
        grid=(TP,),
        in_specs=[
            pl.BlockSpec((1, 1, M),
                         lambda i, m: (_live(i, m), 0, 0)),
            pl.BlockSpec((1, 1, M),
                         lambda i, m: (TP + _live(i, m), 0, 0)),
            pl.BlockSpec((1, Bp, 1), lambda i, m: (0, 0, 0)),
            pl.BlockSpec((1, 2 * Bp, d), lambda i, m: (m[0, i], 0, 0)),
            pl.BlockSpec((1, 1, d), lambda i, m: (m[0, i], 0, 0)),
            pl.BlockSpec((1, d, d), lambda i, m: (m[0, i], 0, 0)),
            pl.BlockSpec((1, 1, d), lambda i, m: (m[0, i], 0, 0)),
        ],
        out_specs=pl.BlockSpec((M, d), lambda i, m: (_live(i, m), 0)),
    )
    out_sorted = pl.pallas_call(
        _mlp_body,
        grid_spec=grid_spec,
        out_shape=jax.ShapeDtypeStruct((Tpad, d), f32),
        compiler_params=pltpu.CompilerParams(
            dimension_semantics=("arbitrary",)),
    )(meta, vals_sorted, vals_sorted, bcol, W1p, b1r, W2, b2r)

    C = 32
    NCH = tpw // C

    @functools.partial(
        pl.kernel, mesh=mesh,
        out_type=jax.ShapeDtypeStruct((T, d), f32),
        scratch_types=[
            pltpu.VMEM((C,), jnp.int32),
            pltpu.VMEM((C,), jnp.int32),
            pltpu.VMEM((C, d), f32),
            pltpu.VMEM((C, d), f32),
            pltpu.SemaphoreType.DMA,
            pltpu.SemaphoreType.DMA,
            pltpu.SemaphoreType.DMA,
            pltpu.SemaphoreType.DMA,
            pltpu.SemaphoreType.DMA,
            pltpu.SemaphoreType.DMA,
        ],
    )
    def gather_rows(table_hbm, pos_hbm, out_hbm, i0, i1, r0, r1,
                    si0, si1, sg0, sg1, sw0, sw1):
        wid = lax.axis_index("s") * NC + lax.axis_index("c")
        base = wid * tpw
        idx = [i0, i1]
        rows = [r0, r1]
        sis = [si0, si1]
        sgs = [sg0, sg1]
        sws = [sw0, sw1]

        di = [None] * NCH
        dg = [None] * NCH
        dw = [None] * NCH
        di[0] = pltpu.async_copy(pos_hbm.at[pl.ds(base, C)], idx[0], sis[0])
        for c in range(NCH):
            b = c % 2
            if c + 1 < NCH:
                di[c + 1] = pltpu.async_copy(
                    pos_hbm.at[pl.ds(base + (c + 1) * C, C)],
                    idx[(c + 1) % 2], sis[(c + 1) % 2])
            di[c].wait()
            if c >= 2:
                dw[c - 2].wait()
            dg[c] = pltpu.async_copy(table_hbm.at[idx[b]], rows[b], sgs[b])
            dg[c].wait()
            dw[c] = pltpu.async_copy(rows[b], out_hbm.at[pl.ds(base + c * C, C)],
                                     sws[b])
        for c in range(max(0, NCH - 2), NCH):
            dw[c].wait()

    out = gather_rows(out_sorted, pos2d.reshape(T))
    return out.reshape(N, S, d)
